# Initial kernel scaffold; baseline (speedup 1.0000x reference)
import jax
import jax.numpy as jnp
from jax.experimental import pallas as pl


def kernel(src1, src2, self_attn1_wq, self_attn1_wk, self_attn1_wv, self_attn1_wo, self_attn1_bq, self_attn1_bk, self_attn1_bv, self_attn1_bo, self_attn2_wq, self_attn2_wk, self_attn2_wv, self_attn2_wo, self_attn2_bq, self_attn2_bk, self_attn2_bv, self_attn2_bo, multihead_attn1_wq, multihead_attn1_wk, multihead_attn1_wv, multihead_attn1_wo, multihead_attn1_bq, multihead_attn1_bk, multihead_attn1_bv, multihead_attn1_bo, multihead_attn2_wq, multihead_attn2_wk, multihead_attn2_wv, multihead_attn2_wo, multihead_attn2_bq, multihead_attn2_bk, multihead_attn2_bv, multihead_attn2_bo, lin11_w, lin11_b, lin12_w, lin12_b, lin21_w, lin21_b, lin22_w, lin22_b, norm11_g, norm11_b, norm12_g, norm12_b, norm13_g, norm13_b, norm21_g, norm21_b, norm22_g, norm22_b, norm23_g, norm23_b):
    raise NotImplementedError("write your pallas kernel here")


# speedup vs baseline: 1.2075x; 1.2075x over previous
import functools
import math

import jax
import jax.numpy as jnp
from jax.experimental import pallas as pl
from jax.experimental.pallas import tpu as pltpu

_EPS = 1e-5


def _rows_ln(y, g, b):
    mean = jnp.mean(y, axis=-1, keepdims=True)
    msq = jnp.mean(y * y, axis=-1, keepdims=True)
    var = msq - mean * mean
    return (y - mean) * jax.lax.rsqrt(var + _EPS) * g + b


def _to_heads(t, bt, s, h, hd):
    return (t.reshape(bt, s, h, hd)
             .transpose(0, 2, 1, 3)
             .reshape(bt * h, s, hd))


def _bmm(a, b_t, dims):
    return jax.lax.dot_general(
        a.astype(jnp.bfloat16), b_t.astype(jnp.bfloat16), dims,
        preferred_element_type=jnp.float32)


_QK_DIMS = (((2,), (2,)), ((0,), (0,)))
_PV_DIMS = (((2,), (1,)), ((0,), (0,)))


def _attention(qh, kh, vh):
    scores = _bmm(qh, kh, _QK_DIMS)
    m = jnp.max(scores, axis=-1, keepdims=True)
    e = jnp.exp(scores - m)
    probs = e / jnp.sum(e, axis=-1, keepdims=True)
    ctx = _bmm(probs, vh, _PV_DIMS)
    return ctx, probs


def _stage1_kernel(x_ref, wqkv_ref, bqkv_ref, wo_ref, bo_ref, g_ref, b_ref,
                   out_ref, *, nhead):
    x = x_ref[0]
    bt, s, d = x.shape
    hd = d // nhead
    x2 = x.reshape(bt * s, d)

    qkv = jnp.dot(x2.astype(jnp.bfloat16), wqkv_ref[0],
                  preferred_element_type=jnp.float32) + bqkv_ref[0]
    qh = _to_heads(qkv[:, :d], bt, s, nhead, hd)
    kh = _to_heads(qkv[:, d:2 * d], bt, s, nhead, hd)
    vh = _to_heads(qkv[:, 2 * d:], bt, s, nhead, hd)

    ctx, _ = _attention(qh, kh, vh)
    ctx = (ctx.reshape(bt, nhead, s, hd)
              .transpose(0, 2, 1, 3)
              .reshape(bt * s, d))
    attn_out = jnp.dot(ctx.astype(jnp.bfloat16), wo_ref[0],
                       preferred_element_type=jnp.float32) + bo_ref[0]

    out = _rows_ln(x2 + attn_out, g_ref[0], b_ref[0])
    out_ref[0] = out.reshape(bt, s, d)


def _stage2_kernel(xq_ref, xkv_ref,
                   wq_ref, bq_ref, wkv_ref, bkv_ref, wo_ref, bo_ref,
                   g1_ref, b1_ref, w1_ref, bf1_ref, w2_ref, bf2_ref,
                   g2_ref, b2_ref,
                   out_ref, attn_ref, *, nhead):
    xq = xq_ref[0]
    xkv = xkv_ref[0]
    bt, s, d = xq.shape
    hd = d // nhead
    q2 = xq.reshape(bt * s, d)
    kv2 = xkv.reshape(bt * s, d)

    qp = jnp.dot(q2.astype(jnp.bfloat16), wq_ref[0],
                 preferred_element_type=jnp.float32) + bq_ref[0]
    kvp = jnp.dot(kv2.astype(jnp.bfloat16), wkv_ref[0],
                  preferred_element_type=jnp.float32) + bkv_ref[0]

    qh = _to_heads(qp, bt, s, nhead, hd)
    kh = _to_heads(kvp[:, :d], bt, s, nhead, hd)
    vh = _to_heads(kvp[:, d:], bt, s, nhead, hd)

    ctx, probs = _attention(qh, kh, vh)
    attn_ref[0] = jnp.mean(probs.reshape(bt, nhead, s, s), axis=1)

    ctx = (ctx.reshape(bt, nhead, s, hd)
              .transpose(0, 2, 1, 3)
              .reshape(bt * s, d))
    attn_out = jnp.dot(ctx.astype(jnp.bfloat16), wo_ref[0],
                       preferred_element_type=jnp.float32) + bo_ref[0]

    y = _rows_ln(q2 + attn_out, g1_ref[0], b1_ref[0])

    h = jnp.dot(y.astype(jnp.bfloat16), w1_ref[0],
                preferred_element_type=jnp.float32) + bf1_ref[0]
    h = jnp.maximum(h, 0.0)
    f = jnp.dot(h.astype(jnp.bfloat16), w2_ref[0],
                preferred_element_type=jnp.float32) + bf2_ref[0]

    out = _rows_ln(y + f, g2_ref[0], b2_ref[0])
    out_ref[0] = out.reshape(bt, s, d)


def _self_params(wq, wk, wv, wo, bq, bk, bv, bo, g, b, nhead):
    d = wq.shape[0]
    scale = 1.0 / math.sqrt(d // nhead)
    wqkv = jnp.concatenate([wq.T * scale, wk.T, wv.T], axis=1).astype(jnp.bfloat16)
    bqkv = jnp.concatenate([bq * scale, bk, bv]).reshape(1, 3 * d)
    return (wqkv, bqkv, wo.T.astype(jnp.bfloat16), bo.reshape(1, d),
            g.reshape(1, d), b.reshape(1, d))


def _cross_params(wq, wk, wv, wo, bq, bk, bv, bo,
                  g1, b1, w1, bf1, w2, bf2, g2, b2, nhead):
    d = wq.shape[0]
    f = w1.shape[0]
    scale = 1.0 / math.sqrt(d // nhead)
    return ((wq.T * scale).astype(jnp.bfloat16), (bq * scale).reshape(1, d),
            jnp.concatenate([wk.T, wv.T], axis=1).astype(jnp.bfloat16),
            jnp.concatenate([bk, bv]).reshape(1, 2 * d),
            wo.T.astype(jnp.bfloat16), bo.reshape(1, d),
            g1.reshape(1, d), b1.reshape(1, d),
            w1.T.astype(jnp.bfloat16), bf1.reshape(1, f),
            w2.T.astype(jnp.bfloat16), bf2.reshape(1, d),
            g2.reshape(1, d), b2.reshape(1, d))


def _wspec(shape):
    nd = len(shape)
    return pl.BlockSpec((1,) + shape[1:], lambda b, j: (b,) + (0,) * (nd - 1))


def kernel(src1, src2, self_attn1_wq, self_attn1_wk, self_attn1_wv, self_attn1_wo, self_attn1_bq, self_attn1_bk, self_attn1_bv, self_attn1_bo, self_attn2_wq, self_attn2_wk, self_attn2_wv, self_attn2_wo, self_attn2_bq, self_attn2_bk, self_attn2_bv, self_attn2_bo, multihead_attn1_wq, multihead_attn1_wk, multihead_attn1_wv, multihead_attn1_wo, multihead_attn1_bq, multihead_attn1_bk, multihead_attn1_bv, multihead_attn1_bo, multihead_attn2_wq, multihead_attn2_wk, multihead_attn2_wv, multihead_attn2_wo, multihead_attn2_bq, multihead_attn2_bk, multihead_attn2_bv, multihead_attn2_bo, lin11_w, lin11_b, lin12_w, lin12_b, lin21_w, lin21_b, lin22_w, lin22_b, norm11_g, norm11_b, norm12_g, norm12_b, norm13_g, norm13_b, norm21_g, norm21_b, norm22_g, norm22_b, norm23_g, norm23_b):
    nhead = 8
    s, b, d = src1.shape
    f = lin11_w.shape[0]
    bt = 8 if b % 8 == 0 else b
    nj = b // bt

    src_stack = jnp.stack([jnp.transpose(src1, (1, 0, 2)),
                           jnp.transpose(src2, (1, 0, 2))], axis=0)

    p1 = [jnp.stack(t) for t in zip(
        _self_params(self_attn1_wq, self_attn1_wk, self_attn1_wv, self_attn1_wo,
                     self_attn1_bq, self_attn1_bk, self_attn1_bv, self_attn1_bo,
                     norm11_g, norm11_b, nhead),
        _self_params(self_attn2_wq, self_attn2_wk, self_attn2_wv, self_attn2_wo,
                     self_attn2_bq, self_attn2_bk, self_attn2_bv, self_attn2_bo,
                     norm21_g, norm21_b, nhead))]

    xspec = pl.BlockSpec((1, bt, s, d), lambda bb, j: (bb, j, 0, 0))
    norm_stack = pl.pallas_call(
        functools.partial(_stage1_kernel, nhead=nhead),
        out_shape=jax.ShapeDtypeStruct((2, b, s, d), jnp.float32),
        grid=(2, nj),
        in_specs=[xspec] + [_wspec(w.shape) for w in p1],
        out_specs=xspec,
        compiler_params=pltpu.CompilerParams(
            dimension_semantics=("parallel", "arbitrary")),
    )(src_stack, *p1)

    p2 = [jnp.stack(t) for t in zip(
        _cross_params(multihead_attn1_wq, multihead_attn1_wk, multihead_attn1_wv,
                      multihead_attn1_wo, multihead_attn1_bq, multihead_attn1_bk,
                      multihead_attn1_bv, multihead_attn1_bo,
                      norm12_g, norm12_b, lin11_w, lin11_b, lin12_w, lin12_b,
                      norm13_g, norm13_b, nhead),
        _cross_params(multihead_attn2_wq, multihead_attn2_wk, multihead_attn2_wv,
                      multihead_attn2_wo, multihead_attn2_bq, multihead_attn2_bk,
                      multihead_attn2_bv, multihead_attn2_bo,
                      norm22_g, norm22_b, lin21_w, lin21_b, lin22_w, lin22_b,
                      norm23_g, norm23_b, nhead))]

    kvspec = pl.BlockSpec((1, bt, s, d), lambda bb, j: (1 - bb, j, 0, 0))
    out_stack, attn_stack = pl.pallas_call(
        functools.partial(_stage2_kernel, nhead=nhead),
        out_shape=(jax.ShapeDtypeStruct((2, b, s, d), jnp.float32),
                   jax.ShapeDtypeStruct((2, b, s, s), jnp.float32)),
        grid=(2, nj),
        in_specs=[xspec, kvspec] + [_wspec(w.shape) for w in p2],
        out_specs=(xspec,
                   pl.BlockSpec((1, bt, s, s), lambda bb, j: (bb, j, 0, 0))),
        compiler_params=pltpu.CompilerParams(
            dimension_semantics=("parallel", "arbitrary")),
    )(norm_stack, norm_stack, *p2)

    out1 = jnp.transpose(out_stack[0], (1, 0, 2))
    out2 = jnp.transpose(out_stack[1], (1, 0, 2))
    return out1, out2, attn_stack[0], attn_stack[1]


# speedup vs baseline: 2.2599x; 1.8715x over previous
import functools

import jax
import jax.numpy as jnp
from jax.experimental import pallas as pl
from jax.experimental.pallas import tpu as pltpu

_EPS = 1e-5
_NHEAD = 8

_TB = (((1,), (1,)), ((), ()))
_QK = (((2,), (2,)), ((0,), (0,)))
_PV = (((2,), (1,)), ((0,), (0,)))


def _dot(a, b, dims):
    return jax.lax.dot_general(a, b, dims, preferred_element_type=jnp.float32)


def _ln(y, g, b):
    mean = jnp.mean(y, axis=-1, keepdims=True)
    msq = jnp.mean(y * y, axis=-1, keepdims=True)
    var = msq - mean * mean
    return (y - mean) * jax.lax.rsqrt(var + _EPS) * g + b


def _heads(t, s, bt):
    d = t.shape[-1]
    hd = d // _NHEAD
    return (t.reshape(s, bt, _NHEAD, hd)
             .transpose(1, 2, 0, 3)
             .reshape(bt * _NHEAD, s, hd))


def _unheads(t, s, bt):
    n, _, hd = t.shape
    return (t.reshape(bt, _NHEAD, s, hd)
             .transpose(2, 0, 1, 3)
             .reshape(s * bt, _NHEAD * hd))


def _softmax(scores):
    e = jnp.exp(scores)
    return e * (1.0 / jnp.sum(e, axis=-1, keepdims=True))


def _self_block(x2, s, bt, scale, wq, bq, wk, bk, wv, bv, wo, bo, g, b):
    q = (_dot(x2, wq, _TB) + bq) * scale
    k = _dot(x2, wk, _TB) + bk
    v = _dot(x2, wv, _TB) + bv
    probs = _softmax(_dot(_heads(q, s, bt), _heads(k, s, bt), _QK))
    ctx = _unheads(_dot(probs, _heads(v, s, bt), _PV), s, bt)
    att = _dot(ctx, wo, _TB) + bo
    return _ln(x2 + att, g, b)


def _cross_ffn_block(y2, ykv, s, bt, scale,
                     wq, bq, wk, bk, wv, bv, wo, bo, g1, b1,
                     w1, bf1, w2, bf2, g2, b2):
    q = (_dot(y2, wq, _TB) + bq) * scale
    k = _dot(ykv, wk, _TB) + bk
    v = _dot(ykv, wv, _TB) + bv
    probs = _softmax(_dot(_heads(q, s, bt), _heads(k, s, bt), _QK))
    ctx = _unheads(_dot(probs, _heads(v, s, bt), _PV), s, bt)
    att = _dot(ctx, wo, _TB) + bo
    y = _ln(y2 + att, g1, b1)
    h = jnp.maximum(_dot(y, w1, _TB) + bf1, 0.0)
    f = _dot(h, w2, _TB) + bf2
    out = _ln(y + f, g2, b2)
    attn = jnp.mean(probs.reshape(bt, _NHEAD, s, s), axis=1)
    return out, attn


def _stage1_kernel(s1_ref, s2_ref,
                   a1wq, a1bq, a1wk, a1bk, a1wv, a1bv, a1wo, a1bo, n11g, n11b,
                   a2wq, a2bq, a2wk, a2bk, a2wv, a2bv, a2wo, a2bo, n21g, n21b,
                   y1_ref, y2_ref, *, scale):
    s, bt, d = s1_ref.shape
    x1 = s1_ref[...].reshape(s * bt, d)
    x2 = s2_ref[...].reshape(s * bt, d)
    y1_ref[...] = _self_block(
        x1, s, bt, scale, a1wq[...], a1bq[...], a1wk[...], a1bk[...],
        a1wv[...], a1bv[...], a1wo[...], a1bo[...], n11g[...], n11b[...]
    ).reshape(s, bt, d)
    y2_ref[...] = _self_block(
        x2, s, bt, scale, a2wq[...], a2bq[...], a2wk[...], a2bk[...],
        a2wv[...], a2bv[...], a2wo[...], a2bo[...], n21g[...], n21b[...]
    ).reshape(s, bt, d)


def _stage2_kernel(y1_ref, y2_ref,
                   c1wq, c1bq, c1wk, c1bk, c1wv, c1bv, c1wo, c1bo, n12g, n12b,
                   l11w, l11b, l12w, l12b, n13g, n13b,
                   c2wq, c2bq, c2wk, c2bk, c2wv, c2bv, c2wo, c2bo, n22g, n22b,
                   l21w, l21b, l22w, l22b, n23g, n23b,
                   out1_ref, out2_ref, attn1_ref, attn2_ref, *, scale):
    s, bt, d = y1_ref.shape
    y1 = y1_ref[...].reshape(s * bt, d)
    y2 = y2_ref[...].reshape(s * bt, d)
    o1, at1 = _cross_ffn_block(
        y1, y2, s, bt, scale,
        c1wq[...], c1bq[...], c1wk[...], c1bk[...], c1wv[...], c1bv[...],
        c1wo[...], c1bo[...], n12g[...], n12b[...],
        l11w[...], l11b[...], l12w[...], l12b[...], n13g[...], n13b[...])
    o2, at2 = _cross_ffn_block(
        y2, y1, s, bt, scale,
        c2wq[...], c2bq[...], c2wk[...], c2bk[...], c2wv[...], c2bv[...],
        c2wo[...], c2bo[...], n22g[...], n22b[...],
        l21w[...], l21b[...], l22w[...], l22b[...], n23g[...], n23b[...])
    out1_ref[...] = o1.reshape(s, bt, d)
    out2_ref[...] = o2.reshape(s, bt, d)
    attn1_ref[...] = at1
    attn2_ref[...] = at2


def _vec(b):
    return b.reshape(1, b.shape[0])


def kernel(src1, src2, self_attn1_wq, self_attn1_wk, self_attn1_wv, self_attn1_wo, self_attn1_bq, self_attn1_bk, self_attn1_bv, self_attn1_bo, self_attn2_wq, self_attn2_wk, self_attn2_wv, self_attn2_wo, self_attn2_bq, self_attn2_bk, self_attn2_bv, self_attn2_bo, multihead_attn1_wq, multihead_attn1_wk, multihead_attn1_wv, multihead_attn1_wo, multihead_attn1_bq, multihead_attn1_bk, multihead_attn1_bv, multihead_attn1_bo, multihead_attn2_wq, multihead_attn2_wk, multihead_attn2_wv, multihead_attn2_wo, multihead_attn2_bq, multihead_attn2_bk, multihead_attn2_bv, multihead_attn2_bo, lin11_w, lin11_b, lin12_w, lin12_b, lin21_w, lin21_b, lin22_w, lin22_b, norm11_g, norm11_b, norm12_g, norm12_b, norm13_g, norm13_b, norm21_g, norm21_b, norm22_g, norm22_b, norm23_g, norm23_b):
    s, b, d = src1.shape
    hd = d // _NHEAD
    scale = 1.0 / (hd ** 0.5)
    bt = 8 if b % 8 == 0 else b
    nj = b // bt

    xspec = pl.BlockSpec((s, bt, d), lambda j: (0, j, 0))

    def wspec(arr):
        nd = arr.ndim
        return pl.BlockSpec(arr.shape, lambda j, _n=nd: (0,) * _n)

    ops1 = [
        src1, src2,
        self_attn1_wq, _vec(self_attn1_bq), self_attn1_wk, _vec(self_attn1_bk),
        self_attn1_wv, _vec(self_attn1_bv), self_attn1_wo, _vec(self_attn1_bo),
        _vec(norm11_g), _vec(norm11_b),
        self_attn2_wq, _vec(self_attn2_bq), self_attn2_wk, _vec(self_attn2_bk),
        self_attn2_wv, _vec(self_attn2_bv), self_attn2_wo, _vec(self_attn2_bo),
        _vec(norm21_g), _vec(norm21_b),
    ]
    y1, y2 = pl.pallas_call(
        functools.partial(_stage1_kernel, scale=scale),
        out_shape=(jax.ShapeDtypeStruct((s, b, d), jnp.float32),
                   jax.ShapeDtypeStruct((s, b, d), jnp.float32)),
        grid=(nj,),
        in_specs=[xspec, xspec] + [wspec(a) for a in ops1[2:]],
        out_specs=(xspec, xspec),
        compiler_params=pltpu.CompilerParams(
            dimension_semantics=("arbitrary",)),
    )(*ops1)

    ops2 = [
        y1, y2,
        multihead_attn1_wq, _vec(multihead_attn1_bq),
        multihead_attn1_wk, _vec(multihead_attn1_bk),
        multihead_attn1_wv, _vec(multihead_attn1_bv),
        multihead_attn1_wo, _vec(multihead_attn1_bo),
        _vec(norm12_g), _vec(norm12_b),
        lin11_w, _vec(lin11_b), lin12_w, _vec(lin12_b),
        _vec(norm13_g), _vec(norm13_b),
        multihead_attn2_wq, _vec(multihead_attn2_bq),
        multihead_attn2_wk, _vec(multihead_attn2_bk),
        multihead_attn2_wv, _vec(multihead_attn2_bv),
        multihead_attn2_wo, _vec(multihead_attn2_bo),
        _vec(norm22_g), _vec(norm22_b),
        lin21_w, _vec(lin21_b), lin22_w, _vec(lin22_b),
        _vec(norm23_g), _vec(norm23_b),
    ]
    out1, out2, attn1, attn2 = pl.pallas_call(
        functools.partial(_stage2_kernel, scale=scale),
        out_shape=(jax.ShapeDtypeStruct((s, b, d), jnp.float32),
                   jax.ShapeDtypeStruct((s, b, d), jnp.float32),
                   jax.ShapeDtypeStruct((b, s, s), jnp.float32),
                   jax.ShapeDtypeStruct((b, s, s), jnp.float32)),
        grid=(nj,),
        in_specs=[xspec, xspec] + [wspec(a) for a in ops2[2:]],
        out_specs=(xspec, xspec,
                   pl.BlockSpec((bt, s, s), lambda j: (j, 0, 0)),
                   pl.BlockSpec((bt, s, s), lambda j: (j, 0, 0))),
        compiler_params=pltpu.CompilerParams(
            dimension_semantics=("arbitrary",)),
    )(*ops2)
    return out1, out2, attn1, attn2


# speedup vs baseline: 2.4397x; 1.0796x over previous
import functools

import jax
import jax.numpy as jnp
from jax.experimental import pallas as pl
from jax.experimental.pallas import tpu as pltpu

_EPS = 1e-5
_NHEAD = 8

_TB = (((1,), (1,)), ((), ()))
_QK = (((2,), (2,)), ((0,), (0,)))
_PV = (((2,), (1,)), ((0,), (0,)))


def _dot(a, b, dims):
    return jax.lax.dot_general(a, b, dims, preferred_element_type=jnp.float32)


def _ln(y, g, b):
    mean = jnp.mean(y, axis=-1, keepdims=True)
    msq = jnp.mean(y * y, axis=-1, keepdims=True)
    var = msq - mean * mean
    return (y - mean) * jax.lax.rsqrt(var + _EPS) * g + b


def _heads(t, s, bt):
    d = t.shape[-1]
    hd = d // _NHEAD
    return (t.reshape(s, bt, _NHEAD, hd)
             .transpose(1, 2, 0, 3)
             .reshape(bt * _NHEAD, s, hd))


def _unheads(t, s, bt):
    n, _, hd = t.shape
    return (t.reshape(bt, _NHEAD, s, hd)
             .transpose(2, 0, 1, 3)
             .reshape(s * bt, _NHEAD * hd))


def _softmax(scores):
    e = jnp.exp(scores)
    return e * (1.0 / jnp.sum(e, axis=-1, keepdims=True))


def _self_block(x2, s, bt, scale, wq, bq, wk, bk, wv, bv, wo, bo, g, b):
    q = (_dot(x2, wq, _TB) + bq) * scale
    k = _dot(x2, wk, _TB) + bk
    v = _dot(x2, wv, _TB) + bv
    probs = _softmax(_dot(_heads(q, s, bt), _heads(k, s, bt), _QK))
    ctx = _unheads(_dot(probs, _heads(v, s, bt), _PV), s, bt)
    att = _dot(ctx, wo, _TB) + bo
    return _ln(x2 + att, g, b)


def _cross_block(y2, ykv, s, bt, scale,
                 wq, bq, wk, bk, wv, bv, wo, bo, g1, b1):
    q = (_dot(y2, wq, _TB) + bq) * scale
    k = _dot(ykv, wk, _TB) + bk
    v = _dot(ykv, wv, _TB) + bv
    probs = _softmax(_dot(_heads(q, s, bt), _heads(k, s, bt), _QK))
    ctx = _unheads(_dot(probs, _heads(v, s, bt), _PV), s, bt)
    att = _dot(ctx, wo, _TB) + bo
    y = _ln(y2 + att, g1, b1)
    attn = jnp.mean(probs.reshape(bt, _NHEAD, s, s), axis=1)
    return y, attn


def _ffn_block(y, w1, bf1, w2, bf2, g2, b2):
    h = jnp.maximum(_dot(y, w1, _TB) + bf1, 0.0)
    f = _dot(h, w2, _TB) + bf2
    return _ln(y + f, g2, b2)


def _stage1_kernel(s1_ref, s2_ref,
                   a1wq, a1bq, a1wk, a1bk, a1wv, a1bv, a1wo, a1bo, n11g, n11b,
                   a2wq, a2bq, a2wk, a2bk, a2wv, a2bv, a2wo, a2bo, n21g, n21b,
                   y1_ref, y2_ref, *, scale):
    s, bt, d = s1_ref.shape
    x1 = s1_ref[...].reshape(s * bt, d)
    x2 = s2_ref[...].reshape(s * bt, d)
    y1_ref[...] = _self_block(
        x1, s, bt, scale, a1wq[...], a1bq[...], a1wk[...], a1bk[...],
        a1wv[...], a1bv[...], a1wo[...], a1bo[...], n11g[...], n11b[...]
    ).reshape(s, bt, d)
    y2_ref[...] = _self_block(
        x2, s, bt, scale, a2wq[...], a2bq[...], a2wk[...], a2bk[...],
        a2wv[...], a2bv[...], a2wo[...], a2bo[...], n21g[...], n21b[...]
    ).reshape(s, bt, d)


def _stage2_kernel(y1_ref, y2_ref,
                   c1wq, c1bq, c1wk, c1bk, c1wv, c1bv, c1wo, c1bo, n12g, n12b,
                   l11w_hbm, l11b, l12w_hbm, l12b, n13g, n13b,
                   c2wq, c2bq, c2wk, c2bk, c2wv, c2bv, c2wo, c2bo, n22g, n22b,
                   l21w_hbm, l21b, l22w_hbm, l22b, n23g, n23b,
                   out1_ref, out2_ref, attn1_ref, attn2_ref,
                   w11_v, w12_v, w21_v, w22_v, ffn_sem, *, scale):
    s, bt, d = y1_ref.shape
    j = pl.program_id(0)

    @pl.when(j == 0)
    def _():
        pltpu.make_async_copy(l11w_hbm, w11_v, ffn_sem.at[0]).start()
        pltpu.make_async_copy(l12w_hbm, w12_v, ffn_sem.at[1]).start()
        pltpu.make_async_copy(l21w_hbm, w21_v, ffn_sem.at[2]).start()
        pltpu.make_async_copy(l22w_hbm, w22_v, ffn_sem.at[3]).start()

    y1 = y1_ref[...].reshape(s * bt, d)
    y2 = y2_ref[...].reshape(s * bt, d)

    m1, at1 = _cross_block(
        y1, y2, s, bt, scale,
        c1wq[...], c1bq[...], c1wk[...], c1bk[...], c1wv[...], c1bv[...],
        c1wo[...], c1bo[...], n12g[...], n12b[...])
    m2, at2 = _cross_block(
        y2, y1, s, bt, scale,
        c2wq[...], c2bq[...], c2wk[...], c2bk[...], c2wv[...], c2bv[...],
        c2wo[...], c2bo[...], n22g[...], n22b[...])

    @pl.when(j == 0)
    def _():
        pltpu.make_async_copy(l11w_hbm, w11_v, ffn_sem.at[0]).wait()
        pltpu.make_async_copy(l12w_hbm, w12_v, ffn_sem.at[1]).wait()
        pltpu.make_async_copy(l21w_hbm, w21_v, ffn_sem.at[2]).wait()
        pltpu.make_async_copy(l22w_hbm, w22_v, ffn_sem.at[3]).wait()

    o1 = _ffn_block(m1, w11_v[...], l11b[...], w12_v[...], l12b[...],
                    n13g[...], n13b[...])
    o2 = _ffn_block(m2, w21_v[...], l21b[...], w22_v[...], l22b[...],
                    n23g[...], n23b[...])
    out1_ref[...] = o1.reshape(s, bt, d)
    out2_ref[...] = o2.reshape(s, bt, d)
    attn1_ref[...] = at1
    attn2_ref[...] = at2


def _vec(b):
    return b.reshape(1, b.shape[0])


def kernel(src1, src2, self_attn1_wq, self_attn1_wk, self_attn1_wv, self_attn1_wo, self_attn1_bq, self_attn1_bk, self_attn1_bv, self_attn1_bo, self_attn2_wq, self_attn2_wk, self_attn2_wv, self_attn2_wo, self_attn2_bq, self_attn2_bk, self_attn2_bv, self_attn2_bo, multihead_attn1_wq, multihead_attn1_wk, multihead_attn1_wv, multihead_attn1_wo, multihead_attn1_bq, multihead_attn1_bk, multihead_attn1_bv, multihead_attn1_bo, multihead_attn2_wq, multihead_attn2_wk, multihead_attn2_wv, multihead_attn2_wo, multihead_attn2_bq, multihead_attn2_bk, multihead_attn2_bv, multihead_attn2_bo, lin11_w, lin11_b, lin12_w, lin12_b, lin21_w, lin21_b, lin22_w, lin22_b, norm11_g, norm11_b, norm12_g, norm12_b, norm13_g, norm13_b, norm21_g, norm21_b, norm22_g, norm22_b, norm23_g, norm23_b):
    s, b, d = src1.shape
    hd = d // _NHEAD
    scale = 1.0 / (hd ** 0.5)
    bt = 8 if b % 8 == 0 else b
    nj = b // bt

    xspec = pl.BlockSpec((s, bt, d), lambda j: (0, j, 0))

    def wspec(arr):
        nd = arr.ndim
        return pl.BlockSpec(arr.shape, lambda j, _n=nd: (0,) * _n)

    ops1 = [
        src1, src2,
        self_attn1_wq, _vec(self_attn1_bq), self_attn1_wk, _vec(self_attn1_bk),
        self_attn1_wv, _vec(self_attn1_bv), self_attn1_wo, _vec(self_attn1_bo),
        _vec(norm11_g), _vec(norm11_b),
        self_attn2_wq, _vec(self_attn2_bq), self_attn2_wk, _vec(self_attn2_bk),
        self_attn2_wv, _vec(self_attn2_bv), self_attn2_wo, _vec(self_attn2_bo),
        _vec(norm21_g), _vec(norm21_b),
    ]
    y1, y2 = pl.pallas_call(
        functools.partial(_stage1_kernel, scale=scale),
        out_shape=(jax.ShapeDtypeStruct((s, b, d), jnp.float32),
                   jax.ShapeDtypeStruct((s, b, d), jnp.float32)),
        grid=(nj,),
        in_specs=[xspec, xspec] + [wspec(a) for a in ops1[2:]],
        out_specs=(xspec, xspec),
        compiler_params=pltpu.CompilerParams(
            dimension_semantics=("arbitrary",)),
    )(*ops1)

    f = lin11_w.shape[0]
    ops2 = [
        y1, y2,
        multihead_attn1_wq, _vec(multihead_attn1_bq),
        multihead_attn1_wk, _vec(multihead_attn1_bk),
        multihead_attn1_wv, _vec(multihead_attn1_bv),
        multihead_attn1_wo, _vec(multihead_attn1_bo),
        _vec(norm12_g), _vec(norm12_b),
        lin11_w, _vec(lin11_b), lin12_w, _vec(lin12_b),
        _vec(norm13_g), _vec(norm13_b),
        multihead_attn2_wq, _vec(multihead_attn2_bq),
        multihead_attn2_wk, _vec(multihead_attn2_bk),
        multihead_attn2_wv, _vec(multihead_attn2_bv),
        multihead_attn2_wo, _vec(multihead_attn2_bo),
        _vec(norm22_g), _vec(norm22_b),
        lin21_w, _vec(lin21_b), lin22_w, _vec(lin22_b),
        _vec(norm23_g), _vec(norm23_b),
    ]
    ffn_names = {12, 14, 28, 30}
    in_specs2 = []
    for i, a in enumerate(ops2):
        if i < 2:
            in_specs2.append(xspec)
        elif i in ffn_names:
            in_specs2.append(pl.BlockSpec(memory_space=pl.ANY))
        else:
            in_specs2.append(wspec(a))
    out1, out2, attn1, attn2 = pl.pallas_call(
        functools.partial(_stage2_kernel, scale=scale),
        out_shape=(jax.ShapeDtypeStruct((s, b, d), jnp.float32),
                   jax.ShapeDtypeStruct((s, b, d), jnp.float32),
                   jax.ShapeDtypeStruct((b, s, s), jnp.float32),
                   jax.ShapeDtypeStruct((b, s, s), jnp.float32)),
        grid=(nj,),
        in_specs=in_specs2,
        out_specs=(xspec, xspec,
                   pl.BlockSpec((bt, s, s), lambda j: (j, 0, 0)),
                   pl.BlockSpec((bt, s, s), lambda j: (j, 0, 0))),
        scratch_shapes=[pltpu.VMEM((f, d), jnp.float32),
                        pltpu.VMEM((d, f), jnp.float32),
                        pltpu.VMEM((f, d), jnp.float32),
                        pltpu.VMEM((d, f), jnp.float32),
                        pltpu.SemaphoreType.DMA((4,))],
        compiler_params=pltpu.CompilerParams(
            dimension_semantics=("arbitrary",)),
    )(*ops2)
    return out1, out2, attn1, attn2
